# Initial kernel scaffold; baseline (speedup 1.0000x reference)
import jax
import jax.numpy as jnp
from jax.experimental import pallas as pl


def kernel(x, edge_index, all_edge_type, W, alpha_table, bias):
    raise NotImplementedError("write your pallas kernel here")


# speedup vs baseline: 11.9120x; 11.9120x over previous
import functools

import jax
import jax.numpy as jnp
from jax import lax
from jax.experimental import pallas as pl
from jax.experimental.pallas import tpu as pltpu
from jax.experimental.pallas import tpu_sc as plsc

K = 128
NW = 32


def _matmul_body(x_ref, w_ref, o_ref):
    o_ref[...] = jnp.dot(x_ref[...], w_ref[...],
                         preferred_element_type=jnp.float32)


def _combine_body(p0_ref, p1_ref, b_ref, o_ref):
    o_ref[...] = p0_ref[...] + p1_ref[...] + b_ref[...]


def _make_sc_kernel(np_, d, epw, nchunk, n_per_tile):
    mesh = plsc.VectorSubcoreMesh(core_axis_name="c", subcore_axis_name="s")

    @functools.partial(
        pl.kernel,
        out_type=jax.ShapeDtypeStruct((2 * np_, d), jnp.float32),
        mesh=mesh,
        compiler_params=pltpu.CompilerParams(needs_layout_passes=False),
        scratch_types=[
            pltpu.VMEM_SHARED((np_, d), jnp.float32),
            pltpu.VMEM((K,), jnp.int32),
            pltpu.VMEM((K,), jnp.int32),
            pltpu.VMEM((K,), jnp.int32),
            pltpu.VMEM((K,), jnp.int32),
            pltpu.VMEM((K, d), jnp.float32),
            pltpu.VMEM((208,), jnp.float32),
            pltpu.SemaphoreType.DMA,
        ],
    )
    def sc_kernel(feats_hbm, src_hbm, dst_hbm, et_hbm, tet_hbm, alpha_hbm,
                  zeros_hbm, out_hbm, acc, src_v, dst_v, et_v, tet_v,
                  rows_v, alpha_v, sem):
        cid = lax.axis_index("c")
        sid = lax.axis_index("s")
        wid = sid * 2 + cid

        pltpu.sync_copy(alpha_hbm, alpha_v)

        zbase = sid * n_per_tile
        pltpu.sync_copy(zeros_hbm.at[pl.ds(zbase, n_per_tile)],
                        acc.at[pl.ds(zbase, n_per_tile)])
        plsc.subcore_barrier()

        ebase = wid * epw

        def chunk_body(i, carry):
            b = ebase + i * K
            pltpu.sync_copy(src_hbm.at[pl.ds(b, K)], src_v)
            pltpu.sync_copy(dst_hbm.at[pl.ds(b, K)], dst_v)
            pltpu.sync_copy(et_hbm.at[pl.ds(b, K)], et_v)
            pltpu.sync_copy(tet_hbm.at[pl.ds(b, K)], tet_v)
            pltpu.async_copy(feats_hbm.at[src_v], rows_v, sem).wait()

            def gbody(g, c):
                e16 = et_v[pl.ds(g * 16, 16)]
                t16 = tet_v[pl.ds(g * 16, 16)]
                a16 = (plsc.load_gather(alpha_v, [e16])
                       + plsc.load_gather(alpha_v, [t16]))
                jbase = g * 16
                for j2 in range(16):
                    s = a16[j2]
                    for h in range(d // 16):
                        rows_v[jbase + j2, pl.ds(h * 16, 16)] = (
                            rows_v[jbase + j2, pl.ds(h * 16, 16)] * s)
                return c

            lax.fori_loop(0, K // 16, gbody, 0)

            pltpu.sync_copy(rows_v, acc.at[dst_v], add=True)
            return carry

        lax.fori_loop(0, nchunk, chunk_body, 0)

        plsc.subcore_barrier()
        pltpu.sync_copy(acc.at[pl.ds(zbase, n_per_tile)],
                        out_hbm.at[pl.ds(cid * np_ + zbase, n_per_tile)])

    return sc_kernel


def kernel(x, edge_index, all_edge_type, W, alpha_table, bias):
    n, d = x.shape
    e = edge_index.shape[1]
    t = (e - n) // 2

    bm = 2000
    feats = pl.pallas_call(
        _matmul_body,
        grid=(n // bm,),
        in_specs=[
            pl.BlockSpec((bm, d), lambda i: (i, 0)),
            pl.BlockSpec((d, d), lambda i: (0, 0)),
        ],
        out_specs=pl.BlockSpec((bm, d), lambda i: (i, 0)),
        out_shape=jax.ShapeDtypeStruct((n, d), jnp.float32),
    )(x, W)

    src = edge_index[0]
    dst = edge_index[1]
    et = all_edge_type
    tet = jnp.concatenate([et[t:2 * t], et[:t], et[2 * t:]])

    ep = ((e + NW * K - 1) // (NW * K)) * (NW * K)
    pad = ep - e
    zi = jnp.zeros((pad,), jnp.int32)
    src_p = jnp.concatenate([src, zi])
    dst_p = jnp.concatenate([dst, zi])
    et_p = jnp.concatenate([et, zi])
    tet_p = jnp.concatenate([tet, zi])

    alpha_flat = jnp.pad(alpha_table[:, 0], (0, 207 - alpha_table.shape[0] + 1))
    np_ = ((n + 16 * 8 - 1) // (16 * 8)) * (16 * 8)
    zeros = jnp.zeros((np_, d), jnp.float32)

    epw = ep // NW
    nchunk = epw // K
    n_per_tile = np_ // 16

    sc_kernel = _make_sc_kernel(np_, d, epw, nchunk, n_per_tile)
    partials = sc_kernel(feats, src_p, dst_p, et_p, tet_p, alpha_flat, zeros)

    p0 = partials[:n]
    p1 = partials[np_:np_ + n]
    out = pl.pallas_call(
        _combine_body,
        grid=(n // bm,),
        in_specs=[
            pl.BlockSpec((bm, d), lambda i: (i, 0)),
            pl.BlockSpec((bm, d), lambda i: (i, 0)),
            pl.BlockSpec((1, d), lambda i: (0, 0)),
        ],
        out_specs=pl.BlockSpec((bm, d), lambda i: (i, 0)),
        out_shape=jax.ShapeDtypeStruct((n, d), jnp.float32),
    )(p0, p1, bias.reshape(1, d))
    return out


# speedup vs baseline: 31.7140x; 2.6624x over previous
import functools

import jax
import jax.numpy as jnp
from jax import lax
from jax.experimental import pallas as pl
from jax.experimental.pallas import tpu as pltpu
from jax.experimental.pallas import tpu_sc as plsc

K = 80
NSC = 2
NWS = 16
NBUF = 5
NSBUF = 2
PD = 2


def _matmul_body(x_ref, w_ref, o_ref):
    o_ref[...] = jnp.dot(x_ref[...], w_ref[...],
                         preferred_element_type=jnp.float32)


def _combine_body(p0_ref, p1_ref, b_ref, o_ref):
    o_ref[...] = (jnp.concatenate([p0_ref[...], p1_ref[...]], axis=1)
                  + b_ref[...])


def _make_sc_kernel(np_, dh, nchunk, n_per_tile):
    mesh = plsc.VectorSubcoreMesh(core_axis_name="c", subcore_axis_name="s")

    @functools.partial(
        pl.kernel,
        out_type=jax.ShapeDtypeStruct((NSC, np_, dh), jnp.float32),
        mesh=mesh,
        compiler_params=pltpu.CompilerParams(needs_layout_passes=False,
                                             use_tc_tiling_on_sc=False),
        scratch_types=(
            [
                pltpu.VMEM_SHARED((np_, dh), jnp.float32),
                pltpu.VMEM((nchunk, 2, K), jnp.int32),
                pltpu.VMEM((208,), jnp.float32),
            ]
            + [pltpu.VMEM((K, dh), jnp.float32) for _ in range(NBUF + NSBUF)]
            + [pltpu.VMEM((K,), jnp.int32) for _ in range(NBUF)]
            + [pltpu.SemaphoreType.DMA for _ in range(NBUF + NSBUF)]
        ),
    )
    def sc_kernel(feats_hbm, pack_hbm, etc_hbm, alpha_hbm, out_hbm,
                  acc, pack_v, alpha_v, *bufs_and_sems):
        rows = bufs_and_sems[:NBUF]
        sbuf = bufs_and_sems[NBUF:NBUF + NSBUF]
        ebuf = bufs_and_sems[NBUF + NSBUF:2 * NBUF + NSBUF]
        gsem = bufs_and_sems[2 * NBUF + NSBUF:3 * NBUF + NSBUF]
        ssem = bufs_and_sems[3 * NBUF + NSBUF:]

        cid = lax.axis_index("c")
        sid = lax.axis_index("s")

        pltpu.sync_copy(alpha_hbm, alpha_v)
        pltpu.sync_copy(pack_hbm.at[sid], pack_v)
        feats_h = feats_hbm.at[cid]

        zv = jnp.zeros((16,), jnp.float32)

        def zbody(j, carry):
            for h in range(dh // 16):
                sbuf[0][j, pl.ds(h * 16, 16)] = zv
            return carry

        lax.fori_loop(0, K, zbody, 0)
        zbase = sid * n_per_tile
        nfull = n_per_tile // K

        def zcopy(q, carry):
            pltpu.sync_copy(sbuf[0], acc.at[pl.ds(zbase + q * K, K)])
            return carry

        lax.fori_loop(0, nfull, zcopy, 0)
        plsc.subcore_barrier()

        ecbase = sid * nchunk * K

        for c0 in range(PD):
            pltpu.async_copy(feats_h.at[pack_v.at[c0, 0]], rows[c0],
                             gsem[c0])
            pltpu.async_copy(etc_hbm.at[pl.ds(ecbase + c0 * K, K)],
                             ebuf[c0], gsem[c0])

        def scale_rows(rin, rout, eb):
            def gbody(g, carry):
                etc16 = eb[pl.ds(g * 16, 16)]
                e16 = lax.bitwise_and(etc16, 0xFFFF)
                t16 = lax.shift_right_logical(etc16, 16)
                a16 = (plsc.load_gather(alpha_v, [e16])
                       + plsc.load_gather(alpha_v, [t16]))
                jbase = g * 16
                for j2 in range(16):
                    s = a16[j2]
                    for h in range(dh // 16):
                        rout[jbase + j2, pl.ds(h * 16, 16)] = (
                            rin[jbase + j2, pl.ds(h * 16, 16)] * s)
                return carry

            lax.fori_loop(0, K // 16, gbody, 0)

        def outer(i, carry):
            for b in range(NBUF):
                c = i * NBUF + b
                cn = c + PD
                bn = (b + PD) % NBUF
                sb = b % NSBUF

                @pl.when(cn < nchunk)
                def _():
                    pltpu.async_copy(feats_h.at[pack_v.at[cn, 0]],
                                     rows[bn], gsem[bn])
                    pltpu.async_copy(etc_hbm.at[pl.ds(ecbase + cn * K, K)],
                                     ebuf[bn], gsem[bn])

                pltpu.make_async_copy(
                    feats_h.at[pack_v.at[c, 0]], rows[b], gsem[b]).wait()
                pltpu.make_async_copy(
                    etc_hbm.at[pl.ds(ecbase + c * K, K)], ebuf[b],
                    gsem[b]).wait()

                @pl.when(c >= NSBUF)
                def _():
                    pltpu.make_async_copy(
                        feats_h.at[pl.ds(0, K)], sbuf[sb],
                        ssem[sb]).wait()

                scale_rows(rows[b], sbuf[sb], ebuf[b])
                pltpu.async_copy(sbuf[sb], acc.at[pack_v.at[c, 1]],
                                 ssem[sb], add=True)
            return carry

        lax.fori_loop(0, nchunk // NBUF, outer, 0)

        for b in range(NSBUF):
            pltpu.make_async_copy(feats_h.at[pl.ds(0, K)], sbuf[b],
                                  ssem[b]).wait()

        plsc.subcore_barrier()

        def wcopy(q, carry):
            pltpu.sync_copy(acc.at[pl.ds(zbase + q * K, K)], rows[0])
            pltpu.sync_copy(rows[0],
                            out_hbm.at[cid, pl.ds(zbase + q * K, K)])
            return carry

        lax.fori_loop(0, nfull, wcopy, 0)

    return sc_kernel


def kernel(x, edge_index, all_edge_type, W, alpha_table, bias):
    n, d = x.shape
    e = edge_index.shape[1]
    t = (e - n) // 2
    dh = d // 2

    bm = 2000
    feats = pl.pallas_call(
        _matmul_body,
        grid=(n // bm,),
        in_specs=[
            pl.BlockSpec((bm, d), lambda i: (i, 0)),
            pl.BlockSpec((d, d), lambda i: (0, 0)),
        ],
        out_specs=pl.BlockSpec((bm, d), lambda i: (i, 0)),
        out_shape=jax.ShapeDtypeStruct((n, d), jnp.float32),
    )(x, W)

    src = edge_index[0]
    dst = edge_index[1]
    et = all_edge_type
    tet = jnp.concatenate([et[t:2 * t], et[:t], et[2 * t:]])

    ep = ((e + NWS * K - 1) // (NWS * K)) * (NWS * K)
    pad = ep - e
    zi = jnp.zeros((pad,), jnp.int32)
    epw = ep // NWS
    nchunk = epw // K
    etc = jnp.concatenate([jnp.bitwise_or(et, jnp.left_shift(tet, 16)), zi])
    pack = jnp.stack([
        jnp.concatenate([src, zi]),
        jnp.concatenate([dst, zi]),
    ])
    pack = pack.reshape(2, NWS, nchunk, K).transpose(1, 2, 0, 3)

    alpha_flat = jnp.pad(alpha_table[:, 0], (0, 207 - alpha_table.shape[0] + 1))
    feats_sp = feats.reshape(n, 2, dh).transpose(1, 0, 2)
    np_ = ((n + NWS * K - 1) // (NWS * K)) * (NWS * K)
    n_per_tile = np_ // NWS

    sc_kernel = _make_sc_kernel(np_, dh, nchunk, n_per_tile)
    partials = sc_kernel(feats_sp, pack, etc, alpha_flat)

    p0 = partials[0, :n]
    p1 = partials[1, :n]
    out = pl.pallas_call(
        _combine_body,
        grid=(n // bm,),
        in_specs=[
            pl.BlockSpec((bm, dh), lambda i: (i, 0)),
            pl.BlockSpec((bm, dh), lambda i: (i, 0)),
            pl.BlockSpec((1, d), lambda i: (0, 0)),
        ],
        out_specs=pl.BlockSpec((bm, d), lambda i: (i, 0)),
        out_shape=jax.ShapeDtypeStruct((n, d), jnp.float32),
    )(p0, p1, bias.reshape(1, d))
    return out


# speedup vs baseline: 35.7365x; 1.1268x over previous
import functools

import jax
import jax.numpy as jnp
from jax import lax
from jax.experimental import pallas as pl
from jax.experimental.pallas import tpu as pltpu
from jax.experimental.pallas import tpu_sc as plsc

K = 80
NSC = 2
NWS = 16
NBUF = 5
NSBUF = 2
PD = 2


def _matmul_body(x_ref, w_ref, o_ref):
    o_ref[0] = jnp.dot(x_ref[...], w_ref[0],
                       preferred_element_type=jnp.float32)


def _combine_body(p_ref, b_ref, o_ref):
    o_ref[...] = (jnp.concatenate([p_ref[0], p_ref[1]], axis=1)
                  + b_ref[...])


def _make_sc_kernel(np_, dh, nchunk, n_per_tile):
    mesh = plsc.VectorSubcoreMesh(core_axis_name="c", subcore_axis_name="s")

    @functools.partial(
        pl.kernel,
        out_type=jax.ShapeDtypeStruct((NSC, np_, dh), jnp.float32),
        mesh=mesh,
        compiler_params=pltpu.CompilerParams(needs_layout_passes=False,
                                             use_tc_tiling_on_sc=False),
        scratch_types=(
            [
                pltpu.VMEM_SHARED((np_, dh), jnp.float32),
                pltpu.VMEM((nchunk, K), jnp.int32),
                pltpu.VMEM((nchunk, K), jnp.int32),
                pltpu.VMEM((208,), jnp.float32),
            ]
            + [pltpu.VMEM((K, dh), jnp.float32) for _ in range(NBUF + NSBUF)]
            + [pltpu.VMEM((K,), jnp.int32) for _ in range(NBUF)]
            + [pltpu.SemaphoreType.DMA for _ in range(NBUF + NSBUF)]
        ),
    )
    def sc_kernel(feats_hbm, src_hbm, dst_hbm, etc_hbm, alpha_hbm, out_hbm,
                  acc, srcv, dstv, alpha_v, *bufs_and_sems):
        rows = bufs_and_sems[:NBUF]
        sbuf = bufs_and_sems[NBUF:NBUF + NSBUF]
        ebuf = bufs_and_sems[NBUF + NSBUF:2 * NBUF + NSBUF]
        gsem = bufs_and_sems[2 * NBUF + NSBUF:3 * NBUF + NSBUF]
        ssem = bufs_and_sems[3 * NBUF + NSBUF:]

        cid = lax.axis_index("c")
        sid = lax.axis_index("s")

        pltpu.sync_copy(alpha_hbm, alpha_v)
        pltpu.sync_copy(src_hbm.at[sid], srcv)
        pltpu.sync_copy(dst_hbm.at[sid], dstv)
        feats_h = feats_hbm.at[cid]

        zv = jnp.zeros((16,), jnp.float32)

        def zbody(j, carry):
            for h in range(dh // 16):
                sbuf[0][j, pl.ds(h * 16, 16)] = zv
            return carry

        lax.fori_loop(0, K, zbody, 0)
        zbase = sid * n_per_tile
        nfull = n_per_tile // K

        def zcopy(q, carry):
            pltpu.sync_copy(sbuf[0], acc.at[pl.ds(zbase + q * K, K)])
            return carry

        lax.fori_loop(0, nfull, zcopy, 0)
        plsc.subcore_barrier()

        ecbase = sid * nchunk * K

        for c0 in range(PD):
            pltpu.async_copy(feats_h.at[srcv.at[c0]], rows[c0],
                             gsem[c0])
            pltpu.async_copy(etc_hbm.at[pl.ds(ecbase + c0 * K, K)],
                             ebuf[c0], gsem[c0])

        def scale_rows(rin, rout, eb):
            def gbody(g, carry):
                etc16 = eb[pl.ds(g * 16, 16)]
                e16 = lax.bitwise_and(etc16, 0xFFFF)
                t16 = lax.shift_right_logical(etc16, 16)
                a16 = (plsc.load_gather(alpha_v, [e16])
                       + plsc.load_gather(alpha_v, [t16]))
                jbase = g * 16
                for j2 in range(16):
                    s = a16[j2]
                    for h in range(dh // 16):
                        rout[jbase + j2, pl.ds(h * 16, 16)] = (
                            rin[jbase + j2, pl.ds(h * 16, 16)] * s)
                return carry

            lax.fori_loop(0, K // 16, gbody, 0)

        def outer(i, carry):
            for b in range(NBUF):
                c = i * NBUF + b
                cn = c + PD
                bn = (b + PD) % NBUF
                sb = b % NSBUF

                @pl.when(cn < nchunk)
                def _():
                    pltpu.async_copy(feats_h.at[srcv.at[cn]],
                                     rows[bn], gsem[bn])
                    pltpu.async_copy(etc_hbm.at[pl.ds(ecbase + cn * K, K)],
                                     ebuf[bn], gsem[bn])

                pltpu.make_async_copy(
                    feats_h.at[srcv.at[c]], rows[b], gsem[b]).wait()
                pltpu.make_async_copy(
                    etc_hbm.at[pl.ds(ecbase + c * K, K)], ebuf[b],
                    gsem[b]).wait()

                @pl.when(c >= NSBUF)
                def _():
                    pltpu.make_async_copy(
                        feats_h.at[pl.ds(0, K)], sbuf[sb],
                        ssem[sb]).wait()

                scale_rows(rows[b], sbuf[sb], ebuf[b])
                pltpu.async_copy(sbuf[sb], acc.at[dstv.at[c]],
                                 ssem[sb], add=True)
            return carry

        lax.fori_loop(0, nchunk // NBUF, outer, 0)

        for b in range(NSBUF):
            pltpu.make_async_copy(feats_h.at[pl.ds(0, K)], sbuf[b],
                                  ssem[b]).wait()

        plsc.subcore_barrier()

        def wcopy(q, carry):
            pltpu.sync_copy(acc.at[pl.ds(zbase + q * K, K)], rows[0])
            pltpu.sync_copy(rows[0],
                            out_hbm.at[cid, pl.ds(zbase + q * K, K)])
            return carry

        lax.fori_loop(0, nfull, wcopy, 0)

    return sc_kernel


def kernel(x, edge_index, all_edge_type, W, alpha_table, bias):
    n, d = x.shape
    e = edge_index.shape[1]
    t = (e - n) // 2
    dh = d // 2

    bm = 2000
    w_sp = W.reshape(d, NSC, dh).transpose(1, 0, 2)
    feats_sp = pl.pallas_call(
        _matmul_body,
        grid=(NSC, n // bm),
        in_specs=[
            pl.BlockSpec((bm, d), lambda j, i: (i, 0)),
            pl.BlockSpec((1, d, dh), lambda j, i: (j, 0, 0)),
        ],
        out_specs=pl.BlockSpec((1, bm, dh), lambda j, i: (j, i, 0)),
        out_shape=jax.ShapeDtypeStruct((NSC, n, dh), jnp.float32),
    )(x, w_sp)

    src = edge_index[0]
    dst = edge_index[1]
    et = all_edge_type
    tet = jnp.concatenate([et[t:2 * t], et[:t], et[2 * t:]])

    ep = ((e + NWS * K - 1) // (NWS * K)) * (NWS * K)
    pad = ep - e
    zi = jnp.zeros((pad,), jnp.int32)
    epw = ep // NWS
    nchunk = epw // K
    etc = jnp.concatenate([jnp.bitwise_or(et, jnp.left_shift(tet, 16)), zi])
    srcp = jnp.concatenate([src, zi]).reshape(NWS, nchunk, K)
    dstp = jnp.concatenate([dst, zi]).reshape(NWS, nchunk, K)

    alpha_flat = jnp.pad(alpha_table[:, 0], (0, 207 - alpha_table.shape[0] + 1))
    np_ = ((n + NWS * K - 1) // (NWS * K)) * (NWS * K)
    n_per_tile = np_ // NWS

    sc_kernel = _make_sc_kernel(np_, dh, nchunk, n_per_tile)
    partials = sc_kernel(feats_sp, srcp, dstp, etc, alpha_flat)

    out = pl.pallas_call(
        _combine_body,
        grid=(n // bm,),
        in_specs=[
            pl.BlockSpec((2, bm, dh), lambda i: (0, i, 0)),
            pl.BlockSpec((1, d), lambda i: (0, 0)),
        ],
        out_specs=pl.BlockSpec((bm, d), lambda i: (i, 0)),
        out_shape=jax.ShapeDtypeStruct((n, d), jnp.float32),
    )(partials, bias.reshape(1, d))
    return out
